# baseline (speedup 1.0000x reference)
import functools

import jax
import jax.numpy as jnp
from jax import lax
from jax.experimental import pallas as pl
from jax.experimental.pallas import tpu as pltpu
from jax.experimental.pallas import tpu_sc as plsc

N = 10000
E = 320000
D = 128
DO = 40
NC = 2
NS = 16
L = 128
CH = 80
EPT = CH * L
E_PAD = NC * NS * EPT
RPS = 632
N_PAD = RPS * NS
CW = 128


def _sc_mesh():
    return plsc.VectorSubcoreMesh(
        core_axis_name="c", subcore_axis_name="s", num_cores=NC, num_subcores=NS)


@functools.lru_cache(maxsize=None)
def _build_agg():
    @functools.partial(
        pl.kernel,
        out_type=jax.ShapeDtypeStruct((NC, N_PAD, D), jnp.float32),
        mesh=_sc_mesh(),
        scratch_types=[
            pltpu.VMEM((CH, L), jnp.int32),
            pltpu.VMEM((CH, L), jnp.int32),
            pltpu.VMEM((L, D), jnp.float32),
            pltpu.VMEM_SHARED((N_PAD, D), jnp.float32),
        ],
    )
    def _agg(x_hbm, src_hbm, dst_hbm, zeros_hbm, out_hbm, src_v, dst_v, gbuf, accum):
        c = lax.axis_index("c")
        s = lax.axis_index("s")
        pltpu.sync_copy(src_hbm.at[c, s], src_v)
        pltpu.sync_copy(dst_hbm.at[c, s], dst_v)
        pltpu.sync_copy(zeros_hbm, accum.at[pl.ds(s * RPS, RPS)])
        plsc.subcore_barrier()

        def body(j, carry):
            pltpu.sync_copy(x_hbm.at[src_v.at[j]], gbuf)
            pltpu.sync_copy(gbuf, accum.at[dst_v.at[j]], add=True)
            return carry

        lax.fori_loop(0, CH, body, 0)
        plsc.subcore_barrier()
        pltpu.sync_copy(accum.at[pl.ds(s * RPS, RPS)],
                        out_hbm.at[c, pl.ds(s * RPS, RPS)])

    return _agg


@functools.lru_cache(maxsize=None)
def _build_deg():
    @functools.partial(
        pl.kernel,
        out_type=jax.ShapeDtypeStruct((NC, N_PAD, CW), jnp.float32),
        mesh=_sc_mesh(),
        scratch_types=[
            pltpu.VMEM((CH, L), jnp.int32),
            pltpu.VMEM((L, CW), jnp.float32),
            pltpu.VMEM_SHARED((N_PAD, CW), jnp.float32),
            pltpu.SemaphoreType.DMA,
        ],
    )
    def _deg(dst_hbm, ones_hbm, zeros_hbm, out_hbm, dst_v, ones_v, accum, sem):
        c = lax.axis_index("c")
        s = lax.axis_index("s")
        pltpu.sync_copy(dst_hbm.at[c, s], dst_v)
        pltpu.sync_copy(ones_hbm, ones_v)
        pltpu.sync_copy(zeros_hbm, accum.at[pl.ds(s * RPS, RPS)])
        plsc.subcore_barrier()

        def fire(j, carry):
            pltpu.async_copy(ones_v, accum.at[dst_v.at[j]], sem, add=True)
            return carry

        lax.fori_loop(0, CH, fire, 0)

        def drain(j, carry):
            pltpu.make_async_copy(ones_v, accum.at[dst_v.at[j]], sem).wait()
            return carry

        lax.fori_loop(0, CH, drain, 0)
        plsc.subcore_barrier()
        pltpu.sync_copy(accum.at[pl.ds(s * RPS, RPS)],
                        out_hbm.at[c, pl.ds(s * RPS, RPS)])

    return _deg


def _layer_head(parts_ref, cnts_ref, x_ref, W_ref, b_ref, g_ref, be_ref):
    cnts = cnts_ref[...]
    cnt = (cnts[0] + cnts[1])[:N, 0]
    inv = 1.0 / jnp.maximum(cnt, 1.0)
    parts = parts_ref[...]
    ssum = parts[0, :N, :] + parts[1, :N, :]
    h = ssum * inv[:, None] + x_ref[...]
    h = jnp.dot(h, W_ref[...], preferred_element_type=jnp.float32) + b_ref[...][None, :]
    mu = jnp.mean(h, axis=0)
    var = jnp.mean(jnp.square(h - mu[None, :]), axis=0)
    h = (h - mu[None, :]) / jnp.sqrt(var[None, :] + 1e-5)
    h = h * g_ref[...][None, :] + be_ref[...][None, :]
    return 0.5 * h * (1.0 + lax.erf(h * 0.7071067811865476))


def _dense_body(parts_ref, cnts_ref, x_ref, W_ref, b_ref, g_ref, be_ref, out_ref):
    out_ref[...] = _layer_head(parts_ref, cnts_ref, x_ref, W_ref, b_ref, g_ref, be_ref)


def _final_body(parts_ref, cnts_ref, x_ref, W_ref, b_ref, g_ref, be_ref,
                Wl_ref, bl_ref, out_ref):
    h = _layer_head(parts_ref, cnts_ref, x_ref, W_ref, b_ref, g_ref, be_ref)
    logits = jnp.dot(h, Wl_ref[...], preferred_element_type=jnp.float32) + bl_ref[...][None, :]
    col = lax.broadcasted_iota(jnp.int32, (N, D), 1)
    logits = jnp.where(col < DO, logits, -1e30)
    m = jnp.max(logits, axis=1, keepdims=True)
    lse = m + jnp.log(jnp.sum(jnp.exp(logits - m), axis=1, keepdims=True))
    out_ref[...] = logits - lse


_dense = pl.pallas_call(
    _dense_body, out_shape=jax.ShapeDtypeStruct((N, D), jnp.float32))
_final = pl.pallas_call(
    _final_body, out_shape=jax.ShapeDtypeStruct((N, D), jnp.float32))


def kernel(x, edge_index, W1, b1, g1, be1, W2, b2, g2, be2, W3, b3, g3, be3, Wl, bl):
    src = edge_index[0]
    dst = edge_index[1]
    srcp = jnp.concatenate(
        [src, jnp.zeros((E_PAD - E,), jnp.int32)]).reshape(NC, NS, CH, L)
    dstp = jnp.concatenate(
        [dst, jnp.full((E_PAD - E,), N, jnp.int32)]).reshape(NC, NS, CH, L)
    zeros_row = jnp.zeros((RPS, D), jnp.float32)
    ones_cnt = jnp.ones((L, CW), jnp.float32)
    Wl_pad = jnp.zeros((D, D), jnp.float32).at[:, :DO].set(Wl)
    bl_pad = jnp.zeros((D,), jnp.float32).at[:DO].set(bl)

    agg = _build_agg()
    cnts = _build_deg()(dstp, ones_cnt, zeros_row)[:, :, :8]
    p = agg(x, srcp, dstp, zeros_row)
    x1 = _dense(p, cnts, x, W1, b1, g1, be1)
    p = agg(x1, srcp, dstp, zeros_row)
    x2 = _dense(p, cnts, x1, W2, b2, g2, be2)
    p = agg(x2, srcp, dstp, zeros_row)
    out = _final(p, cnts, x2, W3, b3, g3, be3, Wl_pad, bl_pad)
    return out[:, :DO]


# speedup vs baseline: 1.0008x; 1.0008x over previous
import functools

import jax
import jax.numpy as jnp
from jax import lax
from jax.experimental import pallas as pl
from jax.experimental.pallas import tpu as pltpu
from jax.experimental.pallas import tpu_sc as plsc

N = 10000
E = 320000
D = 128
DO = 40
NC = 2
NS = 16
L = 128
CH = 80
EPT = CH * L
E_PAD = NC * NS * EPT
RPS = 632
N_PAD = RPS * NS
CW = 128


def _sc_mesh():
    return plsc.VectorSubcoreMesh(
        core_axis_name="c", subcore_axis_name="s", num_cores=NC, num_subcores=NS)


@functools.lru_cache(maxsize=None)
def _build_agg():
    @functools.partial(
        pl.kernel,
        out_type=jax.ShapeDtypeStruct((NC, N_PAD, D), jnp.float32),
        mesh=_sc_mesh(),
        scratch_types=[
            pltpu.VMEM((CH, L), jnp.int32),
            pltpu.VMEM((CH, L), jnp.int32),
            pltpu.VMEM((L, D), jnp.float32),
            pltpu.VMEM_SHARED((N_PAD, D), jnp.float32),
        ],
    )
    def _agg(x_hbm, src_hbm, dst_hbm, zeros_hbm, out_hbm, src_v, dst_v, gbuf, accum):
        c = lax.axis_index("c")
        s = lax.axis_index("s")
        pltpu.sync_copy(src_hbm.at[c, s], src_v)
        pltpu.sync_copy(dst_hbm.at[c, s], dst_v)
        pltpu.sync_copy(zeros_hbm, accum.at[pl.ds(s * RPS, RPS)])
        plsc.subcore_barrier()

        def body(j, carry):
            pltpu.sync_copy(x_hbm.at[src_v.at[j]], gbuf)
            pltpu.sync_copy(gbuf, accum.at[dst_v.at[j]], add=True)
            return carry

        lax.fori_loop(0, CH, body, 0)
        plsc.subcore_barrier()
        pltpu.sync_copy(accum.at[pl.ds(s * RPS, RPS)],
                        out_hbm.at[c, pl.ds(s * RPS, RPS)])

    return _agg


@functools.lru_cache(maxsize=None)
def _build_deg():
    @functools.partial(
        pl.kernel,
        out_type=jax.ShapeDtypeStruct((NC, N_PAD, CW), jnp.float32),
        mesh=_sc_mesh(),
        scratch_types=[
            pltpu.VMEM((CH, L), jnp.int32),
            pltpu.VMEM((L, CW), jnp.float32),
            pltpu.VMEM_SHARED((N_PAD, CW), jnp.float32),
            pltpu.SemaphoreType.DMA,
        ],
    )
    def _deg(dst_hbm, ones_hbm, zeros_hbm, out_hbm, dst_v, ones_v, accum, sem):
        c = lax.axis_index("c")
        s = lax.axis_index("s")
        pltpu.sync_copy(dst_hbm.at[c, s], dst_v)
        pltpu.sync_copy(ones_hbm, ones_v)
        pltpu.sync_copy(zeros_hbm, accum.at[pl.ds(s * RPS, RPS)])
        plsc.subcore_barrier()

        def body(j, carry):
            pltpu.sync_copy(ones_v, accum.at[dst_v.at[j]], add=True)
            return carry

        lax.fori_loop(0, CH, body, 0)
        plsc.subcore_barrier()
        pltpu.sync_copy(accum.at[pl.ds(s * RPS, RPS)],
                        out_hbm.at[c, pl.ds(s * RPS, RPS)])

    return _deg


def _layer_head(parts_ref, cnts_ref, x_ref, W_ref, b_ref, g_ref, be_ref):
    cnts = cnts_ref[...]
    cnt = (cnts[0] + cnts[1])[:N, 0]
    inv = 1.0 / jnp.maximum(cnt, 1.0)
    parts = parts_ref[...]
    ssum = parts[0, :N, :] + parts[1, :N, :]
    h = ssum * inv[:, None] + x_ref[...]
    h = jnp.dot(h, W_ref[...], preferred_element_type=jnp.float32) + b_ref[...][None, :]
    mu = jnp.mean(h, axis=0)
    var = jnp.mean(jnp.square(h - mu[None, :]), axis=0)
    h = (h - mu[None, :]) / jnp.sqrt(var[None, :] + 1e-5)
    h = h * g_ref[...][None, :] + be_ref[...][None, :]
    return 0.5 * h * (1.0 + lax.erf(h * 0.7071067811865476))


def _dense_body(parts_ref, cnts_ref, x_ref, W_ref, b_ref, g_ref, be_ref, out_ref):
    out_ref[...] = _layer_head(parts_ref, cnts_ref, x_ref, W_ref, b_ref, g_ref, be_ref)


def _final_body(parts_ref, cnts_ref, x_ref, W_ref, b_ref, g_ref, be_ref,
                Wl_ref, bl_ref, out_ref):
    h = _layer_head(parts_ref, cnts_ref, x_ref, W_ref, b_ref, g_ref, be_ref)
    logits = jnp.dot(h, Wl_ref[...], preferred_element_type=jnp.float32) + bl_ref[...][None, :]
    col = lax.broadcasted_iota(jnp.int32, (N, D), 1)
    logits = jnp.where(col < DO, logits, -1e30)
    m = jnp.max(logits, axis=1, keepdims=True)
    lse = m + jnp.log(jnp.sum(jnp.exp(logits - m), axis=1, keepdims=True))
    out_ref[...] = logits - lse


_dense = pl.pallas_call(
    _dense_body, out_shape=jax.ShapeDtypeStruct((N, D), jnp.float32))
_final = pl.pallas_call(
    _final_body, out_shape=jax.ShapeDtypeStruct((N, D), jnp.float32))


def kernel(x, edge_index, W1, b1, g1, be1, W2, b2, g2, be2, W3, b3, g3, be3, Wl, bl):
    src = edge_index[0]
    dst = edge_index[1]
    srcp = jnp.concatenate(
        [src, jnp.zeros((E_PAD - E,), jnp.int32)]).reshape(NC, NS, CH, L)
    dstp = jnp.concatenate(
        [dst, jnp.full((E_PAD - E,), N, jnp.int32)]).reshape(NC, NS, CH, L)
    zeros_row = jnp.zeros((RPS, D), jnp.float32)
    ones_cnt = jnp.ones((L, CW), jnp.float32)
    Wl_pad = jnp.zeros((D, D), jnp.float32).at[:, :DO].set(Wl)
    bl_pad = jnp.zeros((D,), jnp.float32).at[:DO].set(bl)

    agg = _build_agg()
    cnts = _build_deg()(dstp, ones_cnt, zeros_row)[:, :, :8]
    p = agg(x, srcp, dstp, zeros_row)
    x1 = _dense(p, cnts, x, W1, b1, g1, be1)
    p = agg(x1, srcp, dstp, zeros_row)
    x2 = _dense(p, cnts, x1, W2, b2, g2, be2)
    p = agg(x2, srcp, dstp, zeros_row)
    out = _final(p, cnts, x2, W3, b3, g3, be3, Wl_pad, bl_pad)
    return out[:, :DO]


# speedup vs baseline: 2.5898x; 2.5878x over previous
import functools

import jax
import jax.numpy as jnp
from jax import lax
from jax.experimental import pallas as pl
from jax.experimental.pallas import tpu as pltpu
from jax.experimental.pallas import tpu_sc as plsc

N = 10000
E = 320000
D = 128
DO = 40
NC = 2
NS = 16
L = 128
CH = 80
EPT = CH * L
E_PAD = NC * NS * EPT
RPS = 632
N_PAD = RPS * NS
CW = 128


def _sc_mesh():
    return plsc.VectorSubcoreMesh(
        core_axis_name="c", subcore_axis_name="s", num_cores=NC, num_subcores=NS)


@functools.lru_cache(maxsize=None)
def _build_agg():
    @functools.partial(
        pl.kernel,
        out_type=jax.ShapeDtypeStruct((NC, N_PAD, D), jnp.float32),
        mesh=_sc_mesh(),
        scratch_types=[
            pltpu.VMEM((CH, L), jnp.int32),
            pltpu.VMEM((CH, L), jnp.int32),
            pltpu.VMEM((L, D), jnp.float32),
            pltpu.VMEM_SHARED((N_PAD, D), jnp.float32),
        ],
    )
    def _agg(x_hbm, src_hbm, dst_hbm, zeros_hbm, out_hbm, src_v, dst_v, gbuf, accum):
        c = lax.axis_index("c")
        s = lax.axis_index("s")
        pltpu.sync_copy(src_hbm.at[c, s], src_v)
        pltpu.sync_copy(dst_hbm.at[c, s], dst_v)
        pltpu.sync_copy(zeros_hbm, accum.at[pl.ds(s * RPS, RPS)])
        plsc.subcore_barrier()

        def body(j, carry):
            pltpu.sync_copy(x_hbm.at[src_v.at[j]], gbuf)
            pltpu.sync_copy(gbuf, accum.at[dst_v.at[j]], add=True)
            return carry

        lax.fori_loop(0, CH, body, 0)
        plsc.subcore_barrier()
        pltpu.sync_copy(accum.at[pl.ds(s * RPS, RPS)],
                        out_hbm.at[c, pl.ds(s * RPS, RPS)])

    return _agg


@functools.lru_cache(maxsize=None)
def _build_deg():
    @functools.partial(
        pl.kernel,
        out_type=jax.ShapeDtypeStruct((NC, N_PAD, CW), jnp.float32),
        mesh=_sc_mesh(),
        scratch_types=[
            pltpu.VMEM((CH, L), jnp.int32),
            pltpu.VMEM((L, CW), jnp.float32),
            pltpu.VMEM_SHARED((N_PAD, CW), jnp.float32),
            pltpu.SemaphoreType.DMA,
        ],
    )
    def _deg(dst_hbm, ones_hbm, zeros_hbm, out_hbm, dst_v, ones_v, accum, sem):
        c = lax.axis_index("c")
        s = lax.axis_index("s")
        pltpu.sync_copy(dst_hbm.at[c, s], dst_v)
        pltpu.sync_copy(ones_hbm, ones_v)
        pltpu.sync_copy(zeros_hbm, accum.at[pl.ds(s * RPS, RPS)])
        plsc.subcore_barrier()

        def body(j, carry):
            pltpu.sync_copy(ones_v, accum.at[dst_v.at[j]], add=True)
            return carry

        lax.fori_loop(0, CH, body, 0)
        plsc.subcore_barrier()
        pltpu.sync_copy(accum.at[pl.ds(s * RPS, RPS)],
                        out_hbm.at[c, pl.ds(s * RPS, RPS)])

    return _deg


def _layer_head(parts_ref, cnts_ref, x_ref, W_ref, b_ref, g_ref, be_ref):
    cnts = cnts_ref[...]
    cnt = (cnts[0] + cnts[1])[:N, 0]
    inv = 1.0 / jnp.maximum(cnt, 1.0)
    parts = parts_ref[...]
    ssum = parts[0, :N, :] + parts[1, :N, :]
    h = ssum * inv[:, None] + x_ref[...]
    h = jnp.dot(h, W_ref[...], preferred_element_type=jnp.float32) + b_ref[...][None, :]
    mu = jnp.mean(h, axis=0)
    var = jnp.mean(jnp.square(h - mu[None, :]), axis=0)
    h = (h - mu[None, :]) / jnp.sqrt(var[None, :] + 1e-5)
    h = h * g_ref[...][None, :] + be_ref[...][None, :]
    return 0.5 * h * (1.0 + lax.erf(h * 0.7071067811865476))


def _dense_body(parts_ref, cnts_ref, x_ref, W_ref, b_ref, g_ref, be_ref, out_ref):
    out_ref[...] = _layer_head(parts_ref, cnts_ref, x_ref, W_ref, b_ref, g_ref, be_ref)


def _final_body(parts_ref, cnts_ref, x_ref, W_ref, b_ref, g_ref, be_ref,
                Wl_ref, bl_ref, out_ref):
    h = _layer_head(parts_ref, cnts_ref, x_ref, W_ref, b_ref, g_ref, be_ref)
    logits = jnp.dot(h, Wl_ref[...], preferred_element_type=jnp.float32) + bl_ref[...][None, :]
    col = lax.broadcasted_iota(jnp.int32, (N, D), 1)
    logits = jnp.where(col < DO, logits, -1e30)
    m = jnp.max(logits, axis=1, keepdims=True)
    lse = m + jnp.log(jnp.sum(jnp.exp(logits - m), axis=1, keepdims=True))
    out_ref[...] = logits - lse


_dense = pl.pallas_call(
    _dense_body, out_shape=jax.ShapeDtypeStruct((N, D), jnp.float32))
_final = pl.pallas_call(
    _final_body, out_shape=jax.ShapeDtypeStruct((N, D), jnp.float32))


def kernel(x, edge_index, W1, b1, g1, be1, W2, b2, g2, be2, W3, b3, g3, be3, Wl, bl):
    src = edge_index[0]
    dst = edge_index[1]
    pad_i = jnp.arange(E_PAD - E, dtype=jnp.int32)
    srcp = jnp.concatenate([src, pad_i % L]).reshape(NC, NS, CH, L)
    dstp = jnp.concatenate(
        [dst, N + pad_i % (N_PAD - N)]).reshape(NC, NS, CH, L)
    zeros_row = jnp.zeros((RPS, D), jnp.float32)
    ones_cnt = jnp.ones((L, CW), jnp.float32)
    Wl_pad = jnp.zeros((D, D), jnp.float32).at[:, :DO].set(Wl)
    bl_pad = jnp.zeros((D,), jnp.float32).at[:DO].set(bl)

    agg = _build_agg()
    cnts = _build_deg()(dstp, ones_cnt, zeros_row)[:, :, :8]
    p = agg(x, srcp, dstp, zeros_row)
    x1 = _dense(p, cnts, x, W1, b1, g1, be1)
    p = agg(x1, srcp, dstp, zeros_row)
    x2 = _dense(p, cnts, x1, W2, b2, g2, be2)
    p = agg(x2, srcp, dstp, zeros_row)
    out = _final(p, cnts, x2, W3, b3, g3, be3, Wl_pad, bl_pad)
    return out[:, :DO]


# speedup vs baseline: 3.7008x; 1.4290x over previous
import functools

import jax
import jax.numpy as jnp
from jax import lax
from jax.experimental import pallas as pl
from jax.experimental.pallas import tpu as pltpu
from jax.experimental.pallas import tpu_sc as plsc

N = 10000
E = 320000
D = 128
DO = 40
NC = 2
NS = 16
L = 128
CH = 80
EPT = CH * L
E_PAD = NC * NS * EPT
RPS = 632
N_PAD = RPS * NS
CW = 128


def _sc_mesh():
    return plsc.VectorSubcoreMesh(
        core_axis_name="c", subcore_axis_name="s", num_cores=NC, num_subcores=NS)


@functools.lru_cache(maxsize=None)
def _build_agg():
    @functools.partial(
        pl.kernel,
        out_type=jax.ShapeDtypeStruct((NC, N_PAD, D), jnp.float32),
        mesh=_sc_mesh(),
        scratch_types=[
            pltpu.VMEM((CH, L), jnp.int32),
            pltpu.VMEM((2, L), jnp.int32),
            pltpu.VMEM((L, D), jnp.float32),
            pltpu.VMEM((L, D), jnp.float32),
            pltpu.VMEM_SHARED((N_PAD, D), jnp.float32),
            pltpu.SemaphoreType.DMA,
            pltpu.SemaphoreType.DMA,
            pltpu.SemaphoreType.DMA,
            pltpu.SemaphoreType.DMA,
        ],
    )
    def _agg(x_hbm, src_hbm, dst_hbm, zeros_hbm, out_hbm,
             src_v, dstb, bufa, bufb, accum, gsa, gsb, dsa, dsb):
        c = lax.axis_index("c")
        s = lax.axis_index("s")
        pltpu.sync_copy(src_hbm.at[c, s], src_v)
        pltpu.sync_copy(zeros_hbm, accum.at[pl.ds(s * RPS, RPS)])
        plsc.subcore_barrier()

        pltpu.async_copy(dst_hbm.at[c, s, 0], dstb.at[0], dsa)
        pltpu.async_copy(dst_hbm.at[c, s, 1], dstb.at[1], dsb)
        pltpu.async_copy(x_hbm.at[src_v.at[0]], bufa, gsa)
        pltpu.async_copy(x_hbm.at[src_v.at[1]], bufb, gsb)

        def body(r, carry):
            j = 2 * r
            pltpu.make_async_copy(x_hbm.at[src_v.at[j]], bufa, gsa).wait()
            pltpu.make_async_copy(dst_hbm.at[c, s, j], dstb.at[0], dsa).wait()
            pltpu.sync_copy(bufa, accum.at[dstb.at[0]], add=True)

            @pl.when(r < CH // 2 - 1)
            def _():
                pltpu.async_copy(dst_hbm.at[c, s, j + 2], dstb.at[0], dsa)
                pltpu.async_copy(x_hbm.at[src_v.at[j + 2]], bufa, gsa)

            pltpu.make_async_copy(x_hbm.at[src_v.at[j + 1]], bufb, gsb).wait()
            pltpu.make_async_copy(dst_hbm.at[c, s, j + 1], dstb.at[1], dsb).wait()
            pltpu.sync_copy(bufb, accum.at[dstb.at[1]], add=True)

            @pl.when(r < CH // 2 - 1)
            def _():
                pltpu.async_copy(dst_hbm.at[c, s, j + 3], dstb.at[1], dsb)
                pltpu.async_copy(x_hbm.at[src_v.at[j + 3]], bufb, gsb)

            return carry

        lax.fori_loop(0, CH // 2, body, 0)
        plsc.subcore_barrier()
        pltpu.sync_copy(accum.at[pl.ds(s * RPS, RPS)],
                        out_hbm.at[c, pl.ds(s * RPS, RPS)])

    return _agg


@functools.lru_cache(maxsize=None)
def _build_deg():
    @functools.partial(
        pl.kernel,
        out_type=jax.ShapeDtypeStruct((NC, N_PAD, CW), jnp.float32),
        mesh=_sc_mesh(),
        scratch_types=[
            pltpu.VMEM((CH, L), jnp.int32),
            pltpu.VMEM((L, CW), jnp.float32),
            pltpu.VMEM_SHARED((N_PAD, CW), jnp.float32),
            pltpu.SemaphoreType.DMA,
        ],
    )
    def _deg(dst_hbm, ones_hbm, zeros_hbm, out_hbm, dst_v, ones_v, accum, sem):
        c = lax.axis_index("c")
        s = lax.axis_index("s")
        pltpu.sync_copy(dst_hbm.at[c, s], dst_v)
        pltpu.sync_copy(ones_hbm, ones_v)
        pltpu.sync_copy(zeros_hbm, accum.at[pl.ds(s * RPS, RPS)])
        plsc.subcore_barrier()

        def body(j, carry):
            pltpu.sync_copy(ones_v, accum.at[dst_v.at[j]], add=True)
            return carry

        lax.fori_loop(0, CH, body, 0)
        plsc.subcore_barrier()
        pltpu.sync_copy(accum.at[pl.ds(s * RPS, RPS)],
                        out_hbm.at[c, pl.ds(s * RPS, RPS)])

    return _deg


def _layer_head(parts_ref, cnts_ref, x_ref, W_ref, b_ref, g_ref, be_ref):
    cnts = cnts_ref[...]
    cnt = (cnts[0] + cnts[1])[:N, 0]
    inv = 1.0 / jnp.maximum(cnt, 1.0)
    parts = parts_ref[...]
    ssum = parts[0, :N, :] + parts[1, :N, :]
    h = ssum * inv[:, None] + x_ref[...]
    h = jnp.dot(h, W_ref[...], preferred_element_type=jnp.float32) + b_ref[...][None, :]
    mu = jnp.mean(h, axis=0)
    var = jnp.mean(jnp.square(h - mu[None, :]), axis=0)
    h = (h - mu[None, :]) / jnp.sqrt(var[None, :] + 1e-5)
    h = h * g_ref[...][None, :] + be_ref[...][None, :]
    return 0.5 * h * (1.0 + lax.erf(h * 0.7071067811865476))


def _dense_body(parts_ref, cnts_ref, x_ref, W_ref, b_ref, g_ref, be_ref, out_ref):
    out_ref[...] = _layer_head(parts_ref, cnts_ref, x_ref, W_ref, b_ref, g_ref, be_ref)


def _final_body(parts_ref, cnts_ref, x_ref, W_ref, b_ref, g_ref, be_ref,
                Wl_ref, bl_ref, out_ref):
    h = _layer_head(parts_ref, cnts_ref, x_ref, W_ref, b_ref, g_ref, be_ref)
    logits = jnp.dot(h, Wl_ref[...], preferred_element_type=jnp.float32) + bl_ref[...][None, :]
    col = lax.broadcasted_iota(jnp.int32, (N, D), 1)
    logits = jnp.where(col < DO, logits, -1e30)
    m = jnp.max(logits, axis=1, keepdims=True)
    lse = m + jnp.log(jnp.sum(jnp.exp(logits - m), axis=1, keepdims=True))
    out_ref[...] = logits - lse


_dense = pl.pallas_call(
    _dense_body, out_shape=jax.ShapeDtypeStruct((N, D), jnp.float32))
_final = pl.pallas_call(
    _final_body, out_shape=jax.ShapeDtypeStruct((N, D), jnp.float32))


def kernel(x, edge_index, W1, b1, g1, be1, W2, b2, g2, be2, W3, b3, g3, be3, Wl, bl):
    src = edge_index[0]
    dst = edge_index[1]
    pad_i = jnp.arange(E_PAD - E, dtype=jnp.int32)
    srcp = jnp.concatenate([src, pad_i % L]).reshape(NC, NS, CH, L)
    dstp = jnp.concatenate(
        [dst, N + pad_i % (N_PAD - N)]).reshape(NC, NS, CH, L)
    zeros_row = jnp.zeros((RPS, D), jnp.float32)
    ones_cnt = jnp.ones((L, CW), jnp.float32)
    Wl_pad = jnp.zeros((D, D), jnp.float32).at[:, :DO].set(Wl)
    bl_pad = jnp.zeros((D,), jnp.float32).at[:DO].set(bl)

    agg = _build_agg()
    cnts = _build_deg()(dstp, ones_cnt, zeros_row)[:, :, :8]
    p = agg(x, srcp, dstp, zeros_row)
    x1 = _dense(p, cnts, x, W1, b1, g1, be1)
    p = agg(x1, srcp, dstp, zeros_row)
    x2 = _dense(p, cnts, x1, W2, b2, g2, be2)
    p = agg(x2, srcp, dstp, zeros_row)
    out = _final(p, cnts, x2, W3, b3, g3, be3, Wl_pad, bl_pad)
    return out[:, :DO]


# speedup vs baseline: 3.7123x; 1.0031x over previous
import functools

import jax
import jax.numpy as jnp
from jax import lax
from jax.experimental import pallas as pl
from jax.experimental.pallas import tpu as pltpu
from jax.experimental.pallas import tpu_sc as plsc

N = 10000
E = 320000
D = 128
DO = 40
NC = 2
NS = 16
L = 128
CH = 80
EPT = CH * L
E_PAD = NC * NS * EPT
RPS = 632
N_PAD = RPS * NS
CW = 128


def _sc_mesh():
    return plsc.VectorSubcoreMesh(
        core_axis_name="c", subcore_axis_name="s", num_cores=NC, num_subcores=NS)


@functools.lru_cache(maxsize=None)
def _build_agg():
    @functools.partial(
        pl.kernel,
        out_type=jax.ShapeDtypeStruct((NC, N_PAD, D), jnp.float32),
        mesh=_sc_mesh(),
        scratch_types=[
            pltpu.VMEM((CH, L), jnp.int32),
            pltpu.VMEM((2, L), jnp.int32),
            pltpu.VMEM((L, D), jnp.float32),
            pltpu.VMEM((L, D), jnp.float32),
            pltpu.VMEM_SHARED((N_PAD, D), jnp.float32),
            pltpu.SemaphoreType.DMA,
            pltpu.SemaphoreType.DMA,
            pltpu.SemaphoreType.DMA,
            pltpu.SemaphoreType.DMA,
        ],
    )
    def _agg(x_hbm, src_hbm, dst_hbm, zeros_hbm, out_hbm,
             src_v, dstb, bufa, bufb, accum, gsa, gsb, dsa, dsb):
        c = lax.axis_index("c")
        s = lax.axis_index("s")
        pltpu.sync_copy(src_hbm.at[c, s], src_v)
        pltpu.sync_copy(zeros_hbm, accum.at[pl.ds(s * RPS, RPS)])
        plsc.subcore_barrier()

        pltpu.async_copy(dst_hbm.at[c, s, 0], dstb.at[0], dsa)
        pltpu.async_copy(dst_hbm.at[c, s, 1], dstb.at[1], dsb)
        pltpu.async_copy(x_hbm.at[src_v.at[0]], bufa, gsa)
        pltpu.async_copy(x_hbm.at[src_v.at[1]], bufb, gsb)

        def body(r, carry):
            j = 2 * r
            pltpu.make_async_copy(x_hbm.at[src_v.at[j]], bufa, gsa).wait()
            pltpu.make_async_copy(dst_hbm.at[c, s, j], dstb.at[0], dsa).wait()
            pltpu.sync_copy(bufa, accum.at[dstb.at[0]], add=True)

            @pl.when(r < CH // 2 - 1)
            def _():
                pltpu.async_copy(dst_hbm.at[c, s, j + 2], dstb.at[0], dsa)
                pltpu.async_copy(x_hbm.at[src_v.at[j + 2]], bufa, gsa)

            pltpu.make_async_copy(x_hbm.at[src_v.at[j + 1]], bufb, gsb).wait()
            pltpu.make_async_copy(dst_hbm.at[c, s, j + 1], dstb.at[1], dsb).wait()
            pltpu.sync_copy(bufb, accum.at[dstb.at[1]], add=True)

            @pl.when(r < CH // 2 - 1)
            def _():
                pltpu.async_copy(dst_hbm.at[c, s, j + 3], dstb.at[1], dsb)
                pltpu.async_copy(x_hbm.at[src_v.at[j + 3]], bufb, gsb)

            return carry

        lax.fori_loop(0, CH // 2, body, 0)
        plsc.subcore_barrier()
        pltpu.sync_copy(accum.at[pl.ds(s * RPS, RPS)],
                        out_hbm.at[c, pl.ds(s * RPS, RPS)])

    return _agg


@functools.lru_cache(maxsize=None)
def _build_deg():
    @functools.partial(
        pl.kernel,
        out_type=jax.ShapeDtypeStruct((NC, N_PAD, CW), jnp.float32),
        mesh=_sc_mesh(),
        scratch_types=[
            pltpu.VMEM((CH, L), jnp.int32),
            pltpu.VMEM((L, CW), jnp.float32),
            pltpu.VMEM_SHARED((N_PAD, CW), jnp.float32),
            pltpu.SemaphoreType.DMA,
        ],
    )
    def _deg(dst_hbm, ones_hbm, zeros_hbm, out_hbm, dst_v, ones_v, accum, sem):
        c = lax.axis_index("c")
        s = lax.axis_index("s")
        pltpu.sync_copy(dst_hbm.at[c, s], dst_v)
        pltpu.sync_copy(ones_hbm, ones_v)
        pltpu.sync_copy(zeros_hbm, accum.at[pl.ds(s * RPS, RPS)])
        plsc.subcore_barrier()

        def fire(j, carry):
            pltpu.async_copy(ones_v, accum.at[dst_v.at[j]], sem, add=True)
            return carry

        lax.fori_loop(0, CH, fire, 0)

        def drain(j, carry):
            pltpu.make_async_copy(ones_v, accum.at[dst_v.at[j]], sem).wait()
            return carry

        lax.fori_loop(0, CH, drain, 0)
        plsc.subcore_barrier()
        pltpu.sync_copy(accum.at[pl.ds(s * RPS, RPS)],
                        out_hbm.at[c, pl.ds(s * RPS, RPS)])

    return _deg


def _layer_head(parts_ref, cnts_ref, x_ref, W_ref, b_ref, g_ref, be_ref):
    cnts = cnts_ref[...]
    cnt = (cnts[0] + cnts[1])[:N, 0]
    inv = 1.0 / jnp.maximum(cnt, 1.0)
    parts = parts_ref[...]
    ssum = parts[0, :N, :] + parts[1, :N, :]
    h = ssum * inv[:, None] + x_ref[...]
    h = jnp.dot(h, W_ref[...], preferred_element_type=jnp.float32) + b_ref[...][None, :]
    mu = jnp.mean(h, axis=0)
    var = jnp.mean(jnp.square(h - mu[None, :]), axis=0)
    h = (h - mu[None, :]) / jnp.sqrt(var[None, :] + 1e-5)
    h = h * g_ref[...][None, :] + be_ref[...][None, :]
    return 0.5 * h * (1.0 + lax.erf(h * 0.7071067811865476))


def _dense_body(parts_ref, cnts_ref, x_ref, W_ref, b_ref, g_ref, be_ref, out_ref):
    out_ref[...] = _layer_head(parts_ref, cnts_ref, x_ref, W_ref, b_ref, g_ref, be_ref)


def _final_body(parts_ref, cnts_ref, x_ref, W_ref, b_ref, g_ref, be_ref,
                Wl_ref, bl_ref, out_ref):
    h = _layer_head(parts_ref, cnts_ref, x_ref, W_ref, b_ref, g_ref, be_ref)
    logits = jnp.dot(h, Wl_ref[...], preferred_element_type=jnp.float32) + bl_ref[...][None, :]
    col = lax.broadcasted_iota(jnp.int32, (N, D), 1)
    logits = jnp.where(col < DO, logits, -1e30)
    m = jnp.max(logits, axis=1, keepdims=True)
    lse = m + jnp.log(jnp.sum(jnp.exp(logits - m), axis=1, keepdims=True))
    out_ref[...] = logits - lse


_dense = pl.pallas_call(
    _dense_body, out_shape=jax.ShapeDtypeStruct((N, D), jnp.float32))
_final = pl.pallas_call(
    _final_body, out_shape=jax.ShapeDtypeStruct((N, D), jnp.float32))


def kernel(x, edge_index, W1, b1, g1, be1, W2, b2, g2, be2, W3, b3, g3, be3, Wl, bl):
    src = edge_index[0]
    dst = edge_index[1]
    pad_i = jnp.arange(E_PAD - E, dtype=jnp.int32)
    srcp = jnp.concatenate([src, pad_i % L]).reshape(NC, NS, CH, L)
    dstp = jnp.concatenate(
        [dst, N + pad_i % (N_PAD - N)]).reshape(NC, NS, CH, L)
    zeros_row = jnp.zeros((RPS, D), jnp.float32)
    ones_cnt = jnp.ones((L, CW), jnp.float32)
    Wl_pad = jnp.zeros((D, D), jnp.float32).at[:, :DO].set(Wl)
    bl_pad = jnp.zeros((D,), jnp.float32).at[:DO].set(bl)

    agg = _build_agg()
    cnts = _build_deg()(dstp, ones_cnt, zeros_row)[:, :, :8]
    p = agg(x, srcp, dstp, zeros_row)
    x1 = _dense(p, cnts, x, W1, b1, g1, be1)
    p = agg(x1, srcp, dstp, zeros_row)
    x2 = _dense(p, cnts, x1, W2, b2, g2, be2)
    p = agg(x2, srcp, dstp, zeros_row)
    out = _final(p, cnts, x2, W3, b3, g3, be3, Wl_pad, bl_pad)
    return out[:, :DO]
